# baseline (speedup 1.0000x reference)
import jax
import jax.numpy as jnp
from jax.experimental import pallas as pl
from jax.experimental.pallas import tpu as pltpu

Q = 1024
K = 100000
D = 64
BK = 2000
STEPS = K // BK
NFULL = BK // 128
TAIL = BK - NFULL * 128
BIG = 3.0e38


def _nn_kernel(q_ref, k_ref, scores_ref, img_ref, idx_ref,
               min_ref, arg_ref, qsq_ref, q2_ref):
    step = pl.program_id(0)

    @pl.when(step == 0)
    def _():
        q0 = q_ref[...]
        qsq_ref[...] = jnp.sum(q0 * q0, axis=1, keepdims=True)
        q2_ref[...] = q0 + q0

    q_sq = qsq_ref[...]
    k = k_ref[...]
    k_sq = jnp.sum(k * k, axis=1)[None, :]
    qk2 = jax.lax.dot_general(
        q2_ref[...], k,
        dimension_numbers=(((1,), (1,)), ((), ())),
        preferred_element_type=jnp.float32,
        precision=jax.lax.Precision.DEFAULT,
    )

    v = (q_sq - qk2[:, 0:128]) + k_sq[:, 0:128]
    c_best = jnp.zeros_like(v)
    for c in range(1, NFULL):
        sl = slice(c * 128, (c + 1) * 128)
        d2c = (q_sq - qk2[:, sl]) + k_sq[:, sl]
        better = d2c < v
        c_best = jnp.where(better, jnp.float32(c), c_best)
        v = jnp.minimum(v, d2c)

    lane = jax.lax.broadcasted_iota(jnp.int32, (Q, 128), 1).astype(jnp.float32)
    j_best = c_best * 128.0 + lane
    m = jnp.min(v, axis=1, keepdims=True)
    j = jnp.min(jnp.where(v <= m, j_best, BIG), axis=1, keepdims=True)

    tsl = slice(NFULL * 128, BK)
    t = (q_sq - qk2[:, tsl]) + k_sq[:, tsl]
    tm = jnp.min(t, axis=1, keepdims=True)
    tlane = jax.lax.broadcasted_iota(jnp.int32, (Q, TAIL), 1).astype(jnp.float32)
    tj = jnp.min(jnp.where(t <= tm, tlane, BIG), axis=1, keepdims=True) \
        + jnp.float32(NFULL * 128)
    tail_better = tm < m
    local_min = jnp.where(tail_better, tm, m)
    local_arg = jnp.where(tail_better, tj, j) + jnp.float32(step * BK)

    @pl.when(step == 0)
    def _():
        min_ref[...] = local_min
        arg_ref[...] = local_arg

    @pl.when(step != 0)
    def _():
        prev = min_ref[...]
        better = local_min < prev
        min_ref[...] = jnp.where(better, local_min, prev)
        arg_ref[...] = jnp.where(better, local_arg, arg_ref[...])

    @pl.when(step == STEPS - 1)
    def _():
        d = jnp.sqrt(jnp.maximum(min_ref[...], 0.0) + 1e-12)
        scores_ref[...] = d
        idx_ref[...] = arg_ref[...].astype(jnp.int32)
        img_ref[...] = jnp.max(d, keepdims=True)


def kernel(queries, keys):
    scores, img, idx = pl.pallas_call(
        _nn_kernel,
        grid=(STEPS,),
        in_specs=[
            pl.BlockSpec((Q, D), lambda i: (0, 0)),
            pl.BlockSpec((BK, D), lambda i: (i, 0)),
        ],
        out_specs=[
            pl.BlockSpec((Q, 1), lambda i: (0, 0)),
            pl.BlockSpec((1, 1), lambda i: (0, 0)),
            pl.BlockSpec((Q, 1), lambda i: (0, 0)),
        ],
        out_shape=[
            jax.ShapeDtypeStruct((Q, 1), jnp.float32),
            jax.ShapeDtypeStruct((1, 1), jnp.float32),
            jax.ShapeDtypeStruct((Q, 1), jnp.int32),
        ],
        scratch_shapes=[
            pltpu.VMEM((Q, 1), jnp.float32),
            pltpu.VMEM((Q, 1), jnp.float32),
            pltpu.VMEM((Q, 1), jnp.float32),
            pltpu.VMEM((Q, D), jnp.float32),
        ],
    )(queries, keys)
    return scores[:, 0], img[0, 0], idx[:, 0]


# speedup vs baseline: 1.1332x; 1.1332x over previous
import jax
import jax.numpy as jnp
from jax.experimental import pallas as pl
from jax.experimental.pallas import tpu as pltpu

Q = 1024
K = 100000
D = 64
BK = 4000
STEPS = K // BK
NFULL = BK // 128
TAIL = BK - NFULL * 128
BIG = 3.0e38


def _nn_kernel(q_ref, k_ref, scores_ref, img_ref, idx_ref,
               min_ref, arg_ref, qsq_ref, q2_ref):
    step = pl.program_id(0)

    @pl.when(step == 0)
    def _():
        q0 = q_ref[...]
        qsq_ref[...] = jnp.sum(q0 * q0, axis=1, keepdims=True)
        q2_ref[...] = q0 + q0

    q_sq = qsq_ref[...]
    k = k_ref[...]
    k_sq = jnp.sum(k * k, axis=1)[None, :]
    qk2 = jax.lax.dot_general(
        q2_ref[...], k,
        dimension_numbers=(((1,), (1,)), ((), ())),
        preferred_element_type=jnp.float32,
        precision=jax.lax.Precision.DEFAULT,
    )

    v = (q_sq - qk2[:, 0:128]) + k_sq[:, 0:128]
    c_best = jnp.zeros_like(v)
    for c in range(1, NFULL):
        sl = slice(c * 128, (c + 1) * 128)
        d2c = (q_sq - qk2[:, sl]) + k_sq[:, sl]
        better = d2c < v
        c_best = jnp.where(better, jnp.float32(c), c_best)
        v = jnp.minimum(v, d2c)

    lane = jax.lax.broadcasted_iota(jnp.int32, (Q, 128), 1).astype(jnp.float32)
    j_best = c_best * 128.0 + lane
    m = jnp.min(v, axis=1, keepdims=True)
    j = jnp.min(jnp.where(v <= m, j_best, BIG), axis=1, keepdims=True)

    tsl = slice(NFULL * 128, BK)
    t = (q_sq - qk2[:, tsl]) + k_sq[:, tsl]
    tm = jnp.min(t, axis=1, keepdims=True)
    tlane = jax.lax.broadcasted_iota(jnp.int32, (Q, TAIL), 1).astype(jnp.float32)
    tj = jnp.min(jnp.where(t <= tm, tlane, BIG), axis=1, keepdims=True) \
        + jnp.float32(NFULL * 128)
    tail_better = tm < m
    local_min = jnp.where(tail_better, tm, m)
    local_arg = jnp.where(tail_better, tj, j) + jnp.float32(step * BK)

    @pl.when(step == 0)
    def _():
        min_ref[...] = local_min
        arg_ref[...] = local_arg

    @pl.when(step != 0)
    def _():
        prev = min_ref[...]
        better = local_min < prev
        min_ref[...] = jnp.where(better, local_min, prev)
        arg_ref[...] = jnp.where(better, local_arg, arg_ref[...])

    @pl.when(step == STEPS - 1)
    def _():
        d = jnp.sqrt(jnp.maximum(min_ref[...], 0.0) + 1e-12)
        scores_ref[...] = d
        idx_ref[...] = arg_ref[...].astype(jnp.int32)
        img_ref[...] = jnp.max(d, keepdims=True)


def kernel(queries, keys):
    scores, img, idx = pl.pallas_call(
        _nn_kernel,
        grid=(STEPS,),
        in_specs=[
            pl.BlockSpec((Q, D), lambda i: (0, 0)),
            pl.BlockSpec((BK, D), lambda i: (i, 0)),
        ],
        out_specs=[
            pl.BlockSpec((Q, 1), lambda i: (0, 0)),
            pl.BlockSpec((1, 1), lambda i: (0, 0)),
            pl.BlockSpec((Q, 1), lambda i: (0, 0)),
        ],
        out_shape=[
            jax.ShapeDtypeStruct((Q, 1), jnp.float32),
            jax.ShapeDtypeStruct((1, 1), jnp.float32),
            jax.ShapeDtypeStruct((Q, 1), jnp.int32),
        ],
        scratch_shapes=[
            pltpu.VMEM((Q, 1), jnp.float32),
            pltpu.VMEM((Q, 1), jnp.float32),
            pltpu.VMEM((Q, 1), jnp.float32),
            pltpu.VMEM((Q, D), jnp.float32),
        ],
    )(queries, keys)
    return scores[:, 0], img[0, 0], idx[:, 0]


# speedup vs baseline: 1.1609x; 1.0244x over previous
import jax
import jax.numpy as jnp
from jax.experimental import pallas as pl
from jax.experimental.pallas import tpu as pltpu

Q = 1024
K = 100000
D = 64
BK = 5000
STEPS = K // BK
NFULL = BK // 128
TAIL = BK - NFULL * 128
BIG = 3.0e38


def _nn_kernel(q_ref, k_ref, scores_ref, img_ref, idx_ref,
               min_ref, arg_ref, qsq_ref, q2_ref):
    step = pl.program_id(0)

    @pl.when(step == 0)
    def _():
        q0 = q_ref[...]
        qsq_ref[...] = jnp.sum(q0 * q0, axis=1, keepdims=True)
        q2_ref[...] = q0 + q0

    q_sq = qsq_ref[...]
    k = k_ref[...]
    k_sq = jnp.sum(k * k, axis=1)[None, :]
    qk2 = jax.lax.dot_general(
        q2_ref[...], k,
        dimension_numbers=(((1,), (1,)), ((), ())),
        preferred_element_type=jnp.float32,
        precision=jax.lax.Precision.DEFAULT,
    )

    v = (q_sq - qk2[:, 0:128]) + k_sq[:, 0:128]
    c_best = jnp.zeros_like(v)
    for c in range(1, NFULL):
        sl = slice(c * 128, (c + 1) * 128)
        d2c = (q_sq - qk2[:, sl]) + k_sq[:, sl]
        better = d2c < v
        c_best = jnp.where(better, jnp.float32(c), c_best)
        v = jnp.minimum(v, d2c)

    lane = jax.lax.broadcasted_iota(jnp.int32, (Q, 128), 1).astype(jnp.float32)
    j_best = c_best * 128.0 + lane
    m = jnp.min(v, axis=1, keepdims=True)
    j = jnp.min(jnp.where(v <= m, j_best, BIG), axis=1, keepdims=True)

    tsl = slice(NFULL * 128, BK)
    t = (q_sq - qk2[:, tsl]) + k_sq[:, tsl]
    tm = jnp.min(t, axis=1, keepdims=True)
    tlane = jax.lax.broadcasted_iota(jnp.int32, (Q, TAIL), 1).astype(jnp.float32)
    tj = jnp.min(jnp.where(t <= tm, tlane, BIG), axis=1, keepdims=True) \
        + jnp.float32(NFULL * 128)
    tail_better = tm < m
    local_min = jnp.where(tail_better, tm, m)
    local_arg = jnp.where(tail_better, tj, j) + jnp.float32(step * BK)

    @pl.when(step == 0)
    def _():
        min_ref[...] = local_min
        arg_ref[...] = local_arg

    @pl.when(step != 0)
    def _():
        prev = min_ref[...]
        better = local_min < prev
        min_ref[...] = jnp.where(better, local_min, prev)
        arg_ref[...] = jnp.where(better, local_arg, arg_ref[...])

    @pl.when(step == STEPS - 1)
    def _():
        d = jnp.sqrt(jnp.maximum(min_ref[...], 0.0) + 1e-12)
        scores_ref[...] = d
        idx_ref[...] = arg_ref[...].astype(jnp.int32)
        img_ref[...] = jnp.max(d, keepdims=True)


def kernel(queries, keys):
    scores, img, idx = pl.pallas_call(
        _nn_kernel,
        grid=(STEPS,),
        in_specs=[
            pl.BlockSpec((Q, D), lambda i: (0, 0)),
            pl.BlockSpec((BK, D), lambda i: (i, 0)),
        ],
        out_specs=[
            pl.BlockSpec((Q, 1), lambda i: (0, 0)),
            pl.BlockSpec((1, 1), lambda i: (0, 0)),
            pl.BlockSpec((Q, 1), lambda i: (0, 0)),
        ],
        out_shape=[
            jax.ShapeDtypeStruct((Q, 1), jnp.float32),
            jax.ShapeDtypeStruct((1, 1), jnp.float32),
            jax.ShapeDtypeStruct((Q, 1), jnp.int32),
        ],
        scratch_shapes=[
            pltpu.VMEM((Q, 1), jnp.float32),
            pltpu.VMEM((Q, 1), jnp.float32),
            pltpu.VMEM((Q, 1), jnp.float32),
            pltpu.VMEM((Q, D), jnp.float32),
        ],
    )(queries, keys)
    return scores[:, 0], img[0, 0], idx[:, 0]


# speedup vs baseline: 1.1900x; 1.0251x over previous
import jax
import jax.numpy as jnp
from jax.experimental import pallas as pl
from jax.experimental.pallas import tpu as pltpu

Q = 1024
K = 100000
D = 64
BK = 10000
STEPS = K // BK
NFULL = BK // 128
TAIL = BK - NFULL * 128
BIG = 3.0e38


def _nn_kernel(q_ref, k_ref, scores_ref, img_ref, idx_ref,
               min_ref, arg_ref, qsq_ref, q2_ref):
    step = pl.program_id(0)

    @pl.when(step == 0)
    def _():
        q0 = q_ref[...]
        qsq_ref[...] = jnp.sum(q0 * q0, axis=1, keepdims=True)
        q2_ref[...] = q0 + q0

    q_sq = qsq_ref[...]
    k = k_ref[...]
    k_sq = jnp.sum(k * k, axis=1)[None, :]
    qk2 = jax.lax.dot_general(
        q2_ref[...], k,
        dimension_numbers=(((1,), (1,)), ((), ())),
        preferred_element_type=jnp.float32,
        precision=jax.lax.Precision.DEFAULT,
    )

    v = (q_sq - qk2[:, 0:128]) + k_sq[:, 0:128]
    c_best = jnp.zeros_like(v)
    for c in range(1, NFULL):
        sl = slice(c * 128, (c + 1) * 128)
        d2c = (q_sq - qk2[:, sl]) + k_sq[:, sl]
        better = d2c < v
        c_best = jnp.where(better, jnp.float32(c), c_best)
        v = jnp.minimum(v, d2c)

    lane = jax.lax.broadcasted_iota(jnp.int32, (Q, 128), 1).astype(jnp.float32)
    j_best = c_best * 128.0 + lane
    m = jnp.min(v, axis=1, keepdims=True)
    j = jnp.min(jnp.where(v <= m, j_best, BIG), axis=1, keepdims=True)

    tsl = slice(NFULL * 128, BK)
    t = (q_sq - qk2[:, tsl]) + k_sq[:, tsl]
    tm = jnp.min(t, axis=1, keepdims=True)
    tlane = jax.lax.broadcasted_iota(jnp.int32, (Q, TAIL), 1).astype(jnp.float32)
    tj = jnp.min(jnp.where(t <= tm, tlane, BIG), axis=1, keepdims=True) \
        + jnp.float32(NFULL * 128)
    tail_better = tm < m
    local_min = jnp.where(tail_better, tm, m)
    local_arg = jnp.where(tail_better, tj, j) + jnp.float32(step * BK)

    @pl.when(step == 0)
    def _():
        min_ref[...] = local_min
        arg_ref[...] = local_arg

    @pl.when(step != 0)
    def _():
        prev = min_ref[...]
        better = local_min < prev
        min_ref[...] = jnp.where(better, local_min, prev)
        arg_ref[...] = jnp.where(better, local_arg, arg_ref[...])

    @pl.when(step == STEPS - 1)
    def _():
        d = jnp.sqrt(jnp.maximum(min_ref[...], 0.0) + 1e-12)
        scores_ref[...] = d
        idx_ref[...] = arg_ref[...].astype(jnp.int32)
        img_ref[...] = jnp.max(d, keepdims=True)


def kernel(queries, keys):
    scores, img, idx = pl.pallas_call(
        _nn_kernel,
        grid=(STEPS,),
        in_specs=[
            pl.BlockSpec((Q, D), lambda i: (0, 0)),
            pl.BlockSpec((BK, D), lambda i: (i, 0)),
        ],
        out_specs=[
            pl.BlockSpec((Q, 1), lambda i: (0, 0)),
            pl.BlockSpec((1, 1), lambda i: (0, 0)),
            pl.BlockSpec((Q, 1), lambda i: (0, 0)),
        ],
        out_shape=[
            jax.ShapeDtypeStruct((Q, 1), jnp.float32),
            jax.ShapeDtypeStruct((1, 1), jnp.float32),
            jax.ShapeDtypeStruct((Q, 1), jnp.int32),
        ],
        scratch_shapes=[
            pltpu.VMEM((Q, 1), jnp.float32),
            pltpu.VMEM((Q, 1), jnp.float32),
            pltpu.VMEM((Q, 1), jnp.float32),
            pltpu.VMEM((Q, D), jnp.float32),
        ],
    )(queries, keys)
    return scores[:, 0], img[0, 0], idx[:, 0]


# speedup vs baseline: 1.2191x; 1.0245x over previous
import jax
import jax.numpy as jnp
from jax.experimental import pallas as pl
from jax.experimental.pallas import tpu as pltpu

Q = 1024
K = 100000
D = 64
BK = 10000
STEPS = K // BK
NFULL = BK // 128
TAIL = BK - NFULL * 128
BIG = 3.0e38


def _nn_kernel(q_ref, k_ref, scores_ref, img_ref, idx_ref,
               min_ref, arg_ref, qsq_ref, q2_ref):
    step = pl.program_id(0)

    @pl.when(step == 0)
    def _():
        q0 = q_ref[...]
        qsq_ref[...] = jnp.sum(q0 * q0, axis=1, keepdims=True)
        q2_ref[...] = q0 + q0

    q_sq = qsq_ref[...]
    k = k_ref[...]
    k_sq = jnp.sum(k * k, axis=1)[None, :]
    qk2 = jax.lax.dot_general(
        q2_ref[...], k,
        dimension_numbers=(((1,), (1,)), ((), ())),
        preferred_element_type=jnp.float32,
        precision=jax.lax.Precision.DEFAULT,
    )

    v = (q_sq - qk2[:, 0:128]) + k_sq[:, 0:128]
    c_best = jnp.zeros_like(v)
    for c in range(1, NFULL):
        sl = slice(c * 128, (c + 1) * 128)
        d2c = (q_sq - qk2[:, sl]) + k_sq[:, sl]
        better = d2c < v
        c_best = jnp.where(better, jnp.float32(c), c_best)
        v = jnp.minimum(v, d2c)

    lane = jax.lax.broadcasted_iota(jnp.int32, (Q, 128), 1).astype(jnp.float32)
    j_best = c_best * 128.0 + lane
    m = jnp.min(v, axis=1, keepdims=True)
    j = jnp.min(jnp.where(v <= m, j_best, BIG), axis=1, keepdims=True)

    tsl = slice(NFULL * 128, BK)
    t = (q_sq - qk2[:, tsl]) + k_sq[:, tsl]
    tm = jnp.min(t, axis=1, keepdims=True)
    tlane = jax.lax.broadcasted_iota(jnp.int32, (Q, TAIL), 1).astype(jnp.float32)
    tj = jnp.min(jnp.where(t <= tm, tlane, BIG), axis=1, keepdims=True) \
        + jnp.float32(NFULL * 128)
    tail_better = tm < m
    local_min = jnp.where(tail_better, tm, m)
    local_arg = jnp.where(tail_better, tj, j) + jnp.float32(step * BK)

    @pl.when(step == 0)
    def _():
        min_ref[...] = local_min
        arg_ref[...] = local_arg

    @pl.when(step != 0)
    def _():
        prev = min_ref[...]
        better = local_min < prev
        min_ref[...] = jnp.where(better, local_min, prev)
        arg_ref[...] = jnp.where(better, local_arg, arg_ref[...])

    @pl.when(step == STEPS - 1)
    def _():
        d = jnp.sqrt(jnp.maximum(min_ref[...], 0.0) + 1e-12)
        scores_ref[...] = d.reshape(1, Q)
        idx_ref[...] = arg_ref[...].astype(jnp.int32).reshape(1, Q)
        img_ref[...] = jnp.max(d, keepdims=True)


def kernel(queries, keys):
    scores, img, idx = pl.pallas_call(
        _nn_kernel,
        grid=(STEPS,),
        in_specs=[
            pl.BlockSpec((Q, D), lambda i: (0, 0)),
            pl.BlockSpec((BK, D), lambda i: (i, 0)),
        ],
        out_specs=[
            pl.BlockSpec((1, Q), lambda i: (0, 0)),
            pl.BlockSpec((1, 1), lambda i: (0, 0)),
            pl.BlockSpec((1, Q), lambda i: (0, 0)),
        ],
        out_shape=[
            jax.ShapeDtypeStruct((1, Q), jnp.float32),
            jax.ShapeDtypeStruct((1, 1), jnp.float32),
            jax.ShapeDtypeStruct((1, Q), jnp.int32),
        ],
        scratch_shapes=[
            pltpu.VMEM((Q, 1), jnp.float32),
            pltpu.VMEM((Q, 1), jnp.float32),
            pltpu.VMEM((Q, 1), jnp.float32),
            pltpu.VMEM((Q, D), jnp.float32),
        ],
    )(queries, keys)
    return scores[0], img[0, 0], idx[0]
